# Initial kernel scaffold; baseline (speedup 1.0000x reference)
import jax
import jax.numpy as jnp
from jax.experimental import pallas as pl


def kernel(x, edge_index, W1, b1, W2, b2, G1w, G1b, G2w, G2b, W4, b4):
    raise NotImplementedError("write your pallas kernel here")


# speedup vs baseline: 6.2730x; 6.2730x over previous
import functools

import jax
import jax.numpy as jnp
from jax import lax
from jax.experimental import pallas as pl
from jax.experimental.pallas import tpu as pltpu
from jax.experimental.pallas import tpu_sc as plsc

_N = 10000
_E = 320000
_NF = 128
_NH = 36
_PAD = 48
_K = 8
_NCLS = 10

_NC = 2
_NS = 16
_NW = _NC * _NS
_EPW = _E // _NW
_CHUNK = 80
_NCHUNK = _EPW // _CHUNK
_NPT = 632


def _stripe_off(sid):
    return pl.multiple_of(
        jnp.where(sid == _NS - 1, _N - _NPT, sid * _NPT), 8)


def _sc_mesh():
    return plsc.VectorSubcoreMesh(core_axis_name="c", subcore_axis_name="s")


def _zero_fill(ref, rows, width):
    z = jnp.zeros((16,), jnp.float32)

    def row(i, _):
        def col(j, _):
            ref[i, pl.ds(j * 16, 16)] = z
            return 0
        return lax.fori_loop(0, width // 16, col, 0)

    lax.fori_loop(0, rows, row, 0)


@functools.cache
def _make_prop(width):
    @functools.partial(
        pl.kernel,
        out_type=jax.ShapeDtypeStruct((_NC, _N, width), jnp.float32),
        mesh=_sc_mesh(),
        compiler_params=pltpu.CompilerParams(use_tc_tiling_on_sc=False),
        scratch_types=[
            pltpu.VMEM((_CHUNK,), jnp.int32),
            pltpu.VMEM((_CHUNK,), jnp.int32),
            pltpu.VMEM((_CHUNK, width), jnp.float32),
            pltpu.VMEM((_NPT, width), jnp.float32),
            pltpu.VMEM_SHARED((_N, width), jnp.float32),
            pltpu.SemaphoreType.DMA,
        ],
    )
    def prop(tab_hbm, src_hbm, dst_hbm, out_hbm,
             src_v, dst_v, rows_v, stripe_v, acc_sh, sem):
        cid = lax.axis_index("c")
        sid = lax.axis_index("s")
        off = _stripe_off(sid)
        _zero_fill(stripe_v, _NPT, width)
        pltpu.sync_copy(stripe_v, acc_sh.at[pl.ds(off, _NPT)])
        plsc.subcore_barrier()

        ebase = (cid * _NS + sid) * _EPW

        def step(t, _):
            b = ebase + t * _CHUNK
            pltpu.sync_copy(src_hbm.at[pl.ds(b, _CHUNK)], src_v)
            pltpu.async_copy(tab_hbm.at[src_v], rows_v, sem).wait()
            pltpu.sync_copy(dst_hbm.at[pl.ds(b, _CHUNK)], dst_v)
            pltpu.sync_copy(rows_v, acc_sh.at[dst_v], add=True)
            return 0

        lax.fori_loop(0, _NCHUNK, step, 0)
        plsc.subcore_barrier()
        pltpu.sync_copy(acc_sh.at[pl.ds(off, _NPT)],
                        out_hbm.at[cid, pl.ds(off, _NPT)])

    return prop


@functools.cache
def _make_degree():
    @functools.partial(
        pl.kernel,
        out_type=jax.ShapeDtypeStruct((_NC, _N, 16), jnp.float32),
        mesh=_sc_mesh(),
        compiler_params=pltpu.CompilerParams(use_tc_tiling_on_sc=False),
        scratch_types=[
            pltpu.VMEM((_CHUNK,), jnp.int32),
            pltpu.VMEM((_CHUNK, 16), jnp.float32),
            pltpu.VMEM((_NPT, 16), jnp.float32),
            pltpu.VMEM_SHARED((_N, 16), jnp.float32),
        ],
    )
    def _sc_degree(src_hbm, out_hbm, src_v, ones_v, stripe_v, acc_sh):
        cid = lax.axis_index("c")
        sid = lax.axis_index("s")
        off = _stripe_off(sid)
        _zero_fill(stripe_v, _NPT, 16)
        pltpu.sync_copy(stripe_v, acc_sh.at[pl.ds(off, _NPT)])
        one = jnp.ones((16,), jnp.float32)

        def orow(i, _):
            ones_v[i, :] = one
            return 0

        lax.fori_loop(0, _CHUNK, orow, 0)
        plsc.subcore_barrier()

        ebase = (cid * _NS + sid) * _EPW

        def step(t, _):
            b = ebase + t * _CHUNK
            pltpu.sync_copy(src_hbm.at[pl.ds(b, _CHUNK)], src_v)
            pltpu.sync_copy(ones_v, acc_sh.at[src_v], add=True)
            return 0

        lax.fori_loop(0, _NCHUNK, step, 0)
        plsc.subcore_barrier()
        pltpu.sync_copy(acc_sh.at[pl.ds(off, _NPT)],
                        out_hbm.at[cid, pl.ds(off, _NPT)])

    return _sc_degree


def _bn(z, eps=1e-5):
    m = jnp.mean(z, axis=0)
    v = jnp.mean((z - m) ** 2, axis=0)
    return (z - m) / jnp.sqrt(v + eps)


def _tc(body, *args, out_shape):
    return pl.pallas_call(body, out_shape=out_shape)(*args)


_MB = 1000


def _tc_dinv_body(degp_ref, dinv_ref):
    degp = degp_ref[...]
    deg = degp[0, :, 0] + degp[1, :, 0]
    dinv_ref[...] = jnp.where(
        deg > 0, lax.rsqrt(jnp.maximum(deg, 1.0)), 0.0)[:, None]


def _tc_proj1_body(x_ref, w1_ref, g1w_ref, dinv_ref, v_ref, xg_ref, tab0_ref):
    x = x_ref[...]
    v1 = jnp.dot(x, w1_ref[...], preferred_element_type=jnp.float32)
    xg = jnp.dot(x, g1w_ref[...], preferred_element_type=jnp.float32)
    dinv = dinv_ref[...]
    v_ref[...] = v1
    xg_ref[...] = xg
    tab0_ref[...] = jnp.concatenate([dinv * v1[:, 7 * _PAD:], xg], axis=1)


def _tc_proj2_body(x11_ref, w2_ref, dinv_ref, v_ref, c27_ref):
    v2 = jnp.dot(x11_ref[...], w2_ref[...], preferred_element_type=jnp.float32)
    v_ref[...] = v2
    c27_ref[...] = dinv_ref[...] * v2[:, 7 * _PAD:]


def _tc_comb1_body(parts_ref, dinv_ref, v6_ref, xg_ref, g1b_ref, g2w_ref,
                   g2b_ref, b6_ref, c6_ref, x2_ref):
    p = parts_ref[...]
    dinv = dinv_ref[...]
    s = p[0, :, :_PAD] + p[1, :, :_PAD]
    agg = p[0, :, _PAD:] + p[1, :, _PAD:]
    b6 = v6_ref[...] - 2.0 * dinv * s
    b6_ref[...] = b6
    c6_ref[...] = dinv * b6
    h = jax.nn.relu(xg_ref[...] + agg + g1b_ref[...])
    g = jnp.dot(h, g2w_ref[...], preferred_element_type=jnp.float32) + g2b_ref[...]
    x2_ref[...] = _bn(jax.nn.relu(g))


def _tc_comb_body(parts_ref, dinv_ref, vprev_ref, bnext_ref, bout_ref, cout_ref):
    p = parts_ref[...]
    dinv = dinv_ref[...]
    b = vprev_ref[...] - 2.0 * dinv * (p[0] + p[1]) - bnext_ref[...]
    bout_ref[...] = b
    cout_ref[...] = dinv * b


def _tc_l1end_body(parts_ref, dinv_ref, v0_ref, b2_ref, bias_ref, x11_ref):
    p = parts_ref[...]
    dinv = dinv_ref[...]
    out1 = v0_ref[...] - dinv * (p[0] + p[1]) - b2_ref[...] + bias_ref[...]
    x11_ref[...] = jax.nn.relu(_bn(out1))


def _tc_l2end_body(parts_ref, dinv_ref, v0_ref, b2_ref, bias_ref, x11_ref,
                   x2_ref, w4_ref, b4_ref, y_ref):
    p = parts_ref[...]
    dinv = dinv_ref[...]
    out2 = v0_ref[...] - dinv * (p[0] + p[1]) - b2_ref[...] + bias_ref[...]
    x12 = jax.nn.relu(_bn(out2))
    x4 = jnp.concatenate(
        [x11_ref[...][:, :_NH], x12[:, :_NH], x2_ref[...][:, :_NH]], axis=1)
    y_ref[...] = jnp.dot(x4, w4_ref[...],
                         preferred_element_type=jnp.float32) + b4_ref[...]


def kernel(x, edge_index, W1, b1, W2, b2, G1w, G1b, G2w, G2b, W4, b4):
    f32 = jnp.float32
    src = edge_index[0]
    dst = edge_index[1]
    pw = _PAD - _NH

    W1c = jnp.pad(W1, ((0, 0), (0, 0), (0, pw))).transpose(1, 0, 2).reshape(_NF, _K * _PAD)
    W2c = jnp.pad(W2, ((0, 0), (0, pw), (0, pw))).transpose(1, 0, 2).reshape(_PAD, _K * _PAD)
    G1wp = jnp.pad(G1w, ((0, 0), (0, pw)))
    G2wp = jnp.pad(G2w, ((0, pw), (0, pw)))
    G1bp = jnp.pad(G1b, (0, pw))[None, :]
    G2bp = jnp.pad(G2b, (0, pw))[None, :]
    b1p = jnp.pad(b1, (0, pw))[None, :]
    b2p = jnp.pad(b2, (0, pw))[None, :]

    sds = jax.ShapeDtypeStruct
    nh48 = sds((_N, _PAD), f32)

    _prop96 = _make_prop(2 * _PAD)
    _prop48 = _make_prop(_PAD)
    degp = _make_degree()(src)
    dinv = _tc(_tc_dinv_body, degp, out_shape=sds((_N, 1), f32))

    ng = _N // _MB
    rb = lambda w: pl.BlockSpec((_MB, w), lambda i: (i, 0))
    full = lambda a: pl.BlockSpec(a.shape, lambda i: (0, 0))
    V1, xg, tab0 = pl.pallas_call(
        _tc_proj1_body, grid=(ng,),
        in_specs=[rb(_NF), full(W1c), full(G1wp), rb(1)],
        out_specs=(rb(_K * _PAD), rb(_PAD), rb(2 * _PAD)),
        out_shape=(sds((_N, _K * _PAD), f32), nh48, sds((_N, 2 * _PAD), f32)),
    )(x, W1c, G1wp, dinv)

    v = lambda V, k: lax.slice_in_dim(V, k * _PAD, (k + 1) * _PAD, axis=1)

    parts = _prop96(tab0, src, dst)
    b6, c, x2 = _tc(_tc_comb1_body, parts, dinv, v(V1, 6), xg, G1bp, G2wp, G2bp,
                    out_shape=(nh48, nh48, nh48))
    bs = {7: v(V1, 7), 6: b6}
    for j in range(6, 1, -1):
        parts = _prop48(c, src, dst)
        bnew, c = _tc(_tc_comb_body, parts, dinv, v(V1, j - 1), bs[j + 1],
                      out_shape=(nh48, nh48))
        bs[j - 1] = bnew
    parts = _prop48(c, src, dst)
    x11 = _tc(_tc_l1end_body, parts, dinv, v(V1, 0), bs[2], b1p,
              out_shape=nh48)
    V2, c = pl.pallas_call(
        _tc_proj2_body, grid=(ng,),
        in_specs=[rb(_PAD), full(W2c), rb(1)],
        out_specs=(rb(_K * _PAD), rb(_PAD)),
        out_shape=(sds((_N, _K * _PAD), f32), nh48),
    )(x11, W2c, dinv)

    parts = _prop48(c, src, dst)
    b6, c = _tc(_tc_comb_body, parts, dinv, v(V2, 6), jnp.zeros((_N, _PAD), f32),
                out_shape=(nh48, nh48))
    bs2 = {7: v(V2, 7), 6: b6}
    for j in range(6, 1, -1):
        parts = _prop48(c, src, dst)
        bnew, c = _tc(_tc_comb_body, parts, dinv, v(V2, j - 1), bs2[j + 1],
                      out_shape=(nh48, nh48))
        bs2[j - 1] = bnew
    parts = _prop48(c, src, dst)
    y = _tc(_tc_l2end_body, parts, dinv, v(V2, 0), bs2[2], b2p, x11, x2,
            W4[0], b4[None, :], out_shape=sds((_N, _NCLS), f32))
    return y


# speedup vs baseline: 15.4680x; 2.4658x over previous
import functools

import jax
import jax.numpy as jnp
from jax import lax
from jax.experimental import pallas as pl
from jax.experimental.pallas import tpu as pltpu
from jax.experimental.pallas import tpu_sc as plsc

_N = 10000
_E = 320000
_NF = 128
_NH = 36
_PAD = 48
_K = 8
_NCLS = 10

_NC = 2
_NS = 16
_NW = _NC * _NS
_EPW = _E // _NW
_CHUNK = 80
_NCHUNK = _EPW // _CHUNK
_G = 5
_NG = _NCHUNK // _G
_NPT = 632


def _stripe_off(sid):
    return pl.multiple_of(
        jnp.where(sid == _NS - 1, _N - _NPT, sid * _NPT), 8)


def _sc_mesh():
    return plsc.VectorSubcoreMesh(core_axis_name="c", subcore_axis_name="s")


def _zero_fill(ref, rows, width):
    z = jnp.zeros((16,), jnp.float32)

    def row(i, _):
        def col(j, _):
            ref[i, pl.ds(j * 16, 16)] = z
            return 0
        return lax.fori_loop(0, width // 16, col, 0)

    lax.fori_loop(0, rows, row, 0)


@functools.cache
def _make_prop(width):
    @functools.partial(
        pl.kernel,
        out_type=jax.ShapeDtypeStruct((_NC, _N, width), jnp.float32),
        mesh=_sc_mesh(),
        compiler_params=pltpu.CompilerParams(use_tc_tiling_on_sc=False),
        scratch_types=[
            pltpu.VMEM((_NCHUNK, _CHUNK), jnp.int32),
            pltpu.VMEM((_NCHUNK, _CHUNK), jnp.int32),
            pltpu.VMEM((_G, _CHUNK, width), jnp.float32),
            pltpu.VMEM((_NPT, width), jnp.float32),
            pltpu.VMEM_SHARED((_N, width), jnp.float32),
            pltpu.SemaphoreType.DMA,
            pltpu.SemaphoreType.DMA,
        ],
    )
    def prop(tab_hbm, src_hbm, dst_hbm, out_hbm,
             src_v, dst_v, bufs_v, stripe_v, acc_sh, gsem, ssem):
        cid = lax.axis_index("c")
        sid = lax.axis_index("s")
        off = _stripe_off(sid)
        wid = cid * _NS + sid
        pltpu.sync_copy(src_hbm.at[wid], src_v)
        pltpu.sync_copy(dst_hbm.at[wid], dst_v)
        _zero_fill(stripe_v, _NPT, width)
        pltpu.sync_copy(stripe_v, acc_sh.at[pl.ds(off, _NPT)])
        plsc.subcore_barrier()

        def group(g, _):
            gd = [pltpu.async_copy(tab_hbm.at[src_v.at[g * _G + b]],
                                   bufs_v.at[b], gsem)
                  for b in range(_G)]
            for d in gd:
                d.wait()
            sd = [pltpu.async_copy(bufs_v.at[b],
                                   acc_sh.at[dst_v.at[g * _G + b]],
                                   ssem, add=True)
                  for b in range(_G)]
            for d in sd:
                d.wait()
            return 0

        lax.fori_loop(0, _NG, group, 0)
        plsc.subcore_barrier()
        pltpu.sync_copy(acc_sh.at[pl.ds(off, _NPT)],
                        out_hbm.at[cid, pl.ds(off, _NPT)])

    return prop


@functools.cache
def _make_degree():
    @functools.partial(
        pl.kernel,
        out_type=jax.ShapeDtypeStruct((_NC, _N, 16), jnp.float32),
        mesh=_sc_mesh(),
        compiler_params=pltpu.CompilerParams(use_tc_tiling_on_sc=False),
        scratch_types=[
            pltpu.VMEM((_NCHUNK, _CHUNK), jnp.int32),
            pltpu.VMEM((_CHUNK, 16), jnp.float32),
            pltpu.VMEM((_NPT, 16), jnp.float32),
            pltpu.VMEM_SHARED((_N, 16), jnp.float32),
            pltpu.SemaphoreType.DMA,
        ],
    )
    def _sc_degree(src_hbm, out_hbm, src_v, ones_v, stripe_v, acc_sh, ssem):
        cid = lax.axis_index("c")
        sid = lax.axis_index("s")
        off = _stripe_off(sid)
        wid = cid * _NS + sid
        pltpu.sync_copy(src_hbm.at[wid], src_v)
        _zero_fill(stripe_v, _NPT, 16)
        pltpu.sync_copy(stripe_v, acc_sh.at[pl.ds(off, _NPT)])
        one = jnp.ones((16,), jnp.float32)

        def orow(i, _):
            ones_v[i, :] = one
            return 0

        lax.fori_loop(0, _CHUNK, orow, 0)
        plsc.subcore_barrier()

        def group(g, _):
            sd = [pltpu.async_copy(ones_v, acc_sh.at[src_v.at[g * _G + b]],
                                   ssem, add=True)
                  for b in range(_G)]
            for d in sd:
                d.wait()
            return 0

        lax.fori_loop(0, _NG, group, 0)
        plsc.subcore_barrier()
        pltpu.sync_copy(acc_sh.at[pl.ds(off, _NPT)],
                        out_hbm.at[cid, pl.ds(off, _NPT)])

    return _sc_degree


def _bn(z, eps=1e-5):
    m = jnp.mean(z, axis=0)
    v = jnp.mean((z - m) ** 2, axis=0)
    return (z - m) / jnp.sqrt(v + eps)


def _tc(body, *args, out_shape):
    return pl.pallas_call(body, out_shape=out_shape)(*args)


_MB = 1000


def _tc_dinv_body(degp_ref, dinv_ref):
    degp = degp_ref[...]
    deg = degp[0, :, 0] + degp[1, :, 0]
    dinv_ref[...] = jnp.where(
        deg > 0, lax.rsqrt(jnp.maximum(deg, 1.0)), 0.0)[:, None]


def _tc_proj1_body(x_ref, w1_ref, g1w_ref, dinv_ref, v_ref, xg_ref, c7_ref):
    x = x_ref[...]
    v1 = jnp.dot(x, w1_ref[...], preferred_element_type=jnp.float32)
    xg = jnp.dot(x, g1w_ref[...], preferred_element_type=jnp.float32)
    v_ref[...] = v1
    xg_ref[...] = xg
    c7_ref[...] = dinv_ref[...] * v1[:, 7 * _PAD:]


def _tc_proj2_body(x11_ref, w2_ref, dinv_ref, v_ref, c27_ref):
    v2 = jnp.dot(x11_ref[...], w2_ref[...], preferred_element_type=jnp.float32)
    v_ref[...] = v2
    c27_ref[...] = dinv_ref[...] * v2[:, 7 * _PAD:]


def _tc_comb1_body(parts_ref, partsg_ref, dinv_ref, v6_ref, xg_ref, g1b_ref,
                   g2w_ref, g2b_ref, b6_ref, c6_ref, x2_ref):
    p = parts_ref[...]
    pg = partsg_ref[...]
    dinv = dinv_ref[...]
    s = p[0] + p[1]
    agg = pg[0] + pg[1]
    b6 = v6_ref[...] - 2.0 * dinv * s
    b6_ref[...] = b6
    c6_ref[...] = dinv * b6
    h = jax.nn.relu(xg_ref[...] + agg + g1b_ref[...])
    g = jnp.dot(h, g2w_ref[...], preferred_element_type=jnp.float32) + g2b_ref[...]
    x2_ref[...] = _bn(jax.nn.relu(g))


def _tc_comb_body(parts_ref, dinv_ref, vprev_ref, bnext_ref, bout_ref, cout_ref):
    p = parts_ref[...]
    dinv = dinv_ref[...]
    b = vprev_ref[...] - 2.0 * dinv * (p[0] + p[1]) - bnext_ref[...]
    bout_ref[...] = b
    cout_ref[...] = dinv * b


def _tc_l1end_body(parts_ref, dinv_ref, v0_ref, b2_ref, bias_ref, x11_ref):
    p = parts_ref[...]
    dinv = dinv_ref[...]
    out1 = v0_ref[...] - dinv * (p[0] + p[1]) - b2_ref[...] + bias_ref[...]
    x11_ref[...] = jax.nn.relu(_bn(out1))


def _tc_l2end_body(parts_ref, dinv_ref, v0_ref, b2_ref, bias_ref, x11_ref,
                   x2_ref, w4_ref, b4_ref, y_ref):
    p = parts_ref[...]
    dinv = dinv_ref[...]
    out2 = v0_ref[...] - dinv * (p[0] + p[1]) - b2_ref[...] + bias_ref[...]
    x12 = jax.nn.relu(_bn(out2))
    x4 = jnp.concatenate(
        [x11_ref[...][:, :_NH], x12[:, :_NH], x2_ref[...][:, :_NH]], axis=1)
    y_ref[...] = jnp.dot(x4, w4_ref[...],
                         preferred_element_type=jnp.float32) + b4_ref[...]


def kernel(x, edge_index, W1, b1, W2, b2, G1w, G1b, G2w, G2b, W4, b4):
    f32 = jnp.float32
    src = edge_index[0]
    dst = edge_index[1]
    pw = _PAD - _NH

    W1c = jnp.pad(W1, ((0, 0), (0, 0), (0, pw))).transpose(1, 0, 2).reshape(_NF, _K * _PAD)
    W2c = jnp.pad(W2, ((0, 0), (0, pw), (0, pw))).transpose(1, 0, 2).reshape(_PAD, _K * _PAD)
    G1wp = jnp.pad(G1w, ((0, 0), (0, pw)))
    G2wp = jnp.pad(G2w, ((0, pw), (0, pw)))
    G1bp = jnp.pad(G1b, (0, pw))[None, :]
    G2bp = jnp.pad(G2b, (0, pw))[None, :]
    b1p = jnp.pad(b1, (0, pw))[None, :]
    b2p = jnp.pad(b2, (0, pw))[None, :]

    sds = jax.ShapeDtypeStruct
    nh48 = sds((_N, _PAD), f32)

    _prop48 = _make_prop(_PAD)
    src3 = src.reshape(_NW, _NCHUNK, _CHUNK)
    dst3 = dst.reshape(_NW, _NCHUNK, _CHUNK)
    degp = _make_degree()(src3)
    dinv = _tc(_tc_dinv_body, degp, out_shape=sds((_N, 1), f32))

    ng = _N // _MB
    rb = lambda w: pl.BlockSpec((_MB, w), lambda i: (i, 0))
    full = lambda a: pl.BlockSpec(a.shape, lambda i: (0, 0))
    V1, xg, c = pl.pallas_call(
        _tc_proj1_body, grid=(ng,),
        in_specs=[rb(_NF), full(W1c), full(G1wp), rb(1)],
        out_specs=(rb(_K * _PAD), rb(_PAD), rb(_PAD)),
        out_shape=(sds((_N, _K * _PAD), f32), nh48, nh48),
    )(x, W1c, G1wp, dinv)

    v = lambda V, k: lax.slice_in_dim(V, k * _PAD, (k + 1) * _PAD, axis=1)

    parts = _prop48(c, src3, dst3)
    partsg = _prop48(xg, src3, dst3)
    b6, c, x2 = _tc(_tc_comb1_body, parts, partsg, dinv, v(V1, 6), xg,
                    G1bp, G2wp, G2bp, out_shape=(nh48, nh48, nh48))
    bs = {7: v(V1, 7), 6: b6}
    for j in range(6, 1, -1):
        parts = _prop48(c, src3, dst3)
        bnew, c = _tc(_tc_comb_body, parts, dinv, v(V1, j - 1), bs[j + 1],
                      out_shape=(nh48, nh48))
        bs[j - 1] = bnew
    parts = _prop48(c, src3, dst3)
    x11 = _tc(_tc_l1end_body, parts, dinv, v(V1, 0), bs[2], b1p,
              out_shape=nh48)
    V2, c = pl.pallas_call(
        _tc_proj2_body, grid=(ng,),
        in_specs=[rb(_PAD), full(W2c), rb(1)],
        out_specs=(rb(_K * _PAD), rb(_PAD)),
        out_shape=(sds((_N, _K * _PAD), f32), nh48),
    )(x11, W2c, dinv)

    parts = _prop48(c, src3, dst3)
    b6, c = _tc(_tc_comb_body, parts, dinv, v(V2, 6), jnp.zeros((_N, _PAD), f32),
                out_shape=(nh48, nh48))
    bs2 = {7: v(V2, 7), 6: b6}
    for j in range(6, 1, -1):
        parts = _prop48(c, src3, dst3)
        bnew, c = _tc(_tc_comb_body, parts, dinv, v(V2, j - 1), bs2[j + 1],
                      out_shape=(nh48, nh48))
        bs2[j - 1] = bnew
    parts = _prop48(c, src3, dst3)
    y = _tc(_tc_l2end_body, parts, dinv, v(V2, 0), bs2[2], b2p, x11, x2,
            W4[0], b4[None, :], out_shape=sds((_N, _NCLS), f32))
    return y


# speedup vs baseline: 20.5691x; 1.3298x over previous
import functools

import jax
import jax.numpy as jnp
from jax import lax
from jax.experimental import pallas as pl
from jax.experimental.pallas import tpu as pltpu
from jax.experimental.pallas import tpu_sc as plsc

_N = 10000
_E = 320000
_NF = 128
_NH = 36
_PAD = 48
_K = 8
_NCLS = 10

_NC = 2
_NS = 16
_NW = _NC * _NS
_EPW = _E // _NW
_CHUNK = 80
_NCHUNK = _EPW // _CHUNK
_G = 5
_NG = _NCHUNK // _G
_NSLOT = 10
_LAG = 5
_NPT = 632


def _stripe_off(sid):
    return pl.multiple_of(
        jnp.where(sid == _NS - 1, _N - _NPT, sid * _NPT), 8)


def _sc_mesh():
    return plsc.VectorSubcoreMesh(core_axis_name="c", subcore_axis_name="s")


def _zero_fill(ref, rows, width):
    z = jnp.zeros((16,), jnp.float32)

    def row(i, _):
        def col(j, _):
            ref[i, pl.ds(j * 16, 16)] = z
            return 0
        return lax.fori_loop(0, width // 16, col, 0)

    lax.fori_loop(0, rows, row, 0)


@functools.cache
def _make_prop(width):
    @functools.partial(
        pl.kernel,
        out_type=jax.ShapeDtypeStruct((_NC, _N, width), jnp.float32),
        mesh=_sc_mesh(),
        compiler_params=pltpu.CompilerParams(use_tc_tiling_on_sc=False),
        scratch_types=[
            pltpu.VMEM((_NCHUNK, _CHUNK), jnp.int32),
            pltpu.VMEM((_NCHUNK, _CHUNK), jnp.int32),
            pltpu.VMEM((_NSLOT, _CHUNK, width), jnp.float32),
            pltpu.VMEM((_NPT, width), jnp.float32),
            pltpu.VMEM_SHARED((_N, width), jnp.float32),
            pltpu.SemaphoreType.DMA,
            pltpu.SemaphoreType.DMA,
        ],
    )
    def prop(tab_hbm, src_hbm, dst_hbm, out_hbm,
             src_v, dst_v, bufs_v, stripe_v, acc_sh, gsem, ssem):
        cid = lax.axis_index("c")
        sid = lax.axis_index("s")
        off = _stripe_off(sid)
        wid = cid * _NS + sid
        pltpu.sync_copy(src_hbm.at[wid], src_v)
        pltpu.sync_copy(dst_hbm.at[wid], dst_v)
        _zero_fill(stripe_v, _NPT, width)
        pltpu.sync_copy(stripe_v, acc_sh.at[pl.ds(off, _NPT)])
        plsc.subcore_barrier()

        def gather(j):
            return pltpu.make_async_copy(
                tab_hbm.at[src_v.at[j]],
                bufs_v.at[lax.rem(j, _NSLOT)], gsem)

        def scatter(j):
            return pltpu.make_async_copy(
                bufs_v.at[lax.rem(j, _NSLOT)],
                acc_sh.at[dst_v.at[j]], ssem)

        for j in range(_LAG):
            gather(j).start()

        def step(j, _):
            gather(j).wait()
            scatter(j).start(add=True)
            pl.when(j >= _LAG)(lambda: scatter(j - _LAG).wait())
            pl.when(j < _NCHUNK - _LAG)(lambda: gather(j + _LAG).start())
            return 0

        lax.fori_loop(0, _NCHUNK, step, 0)

        def tail(t, _):
            scatter(_NCHUNK - _LAG + t).wait()
            return 0

        lax.fori_loop(0, _LAG, tail, 0)
        plsc.subcore_barrier()
        pltpu.sync_copy(acc_sh.at[pl.ds(off, _NPT)],
                        out_hbm.at[cid, pl.ds(off, _NPT)])

    return prop


@functools.cache
def _make_degree():
    @functools.partial(
        pl.kernel,
        out_type=jax.ShapeDtypeStruct((_NC, _N, 16), jnp.float32),
        mesh=_sc_mesh(),
        compiler_params=pltpu.CompilerParams(use_tc_tiling_on_sc=False),
        scratch_types=[
            pltpu.VMEM((_NCHUNK, _CHUNK), jnp.int32),
            pltpu.VMEM((_CHUNK, 16), jnp.float32),
            pltpu.VMEM((_NPT, 16), jnp.float32),
            pltpu.VMEM_SHARED((_N, 16), jnp.float32),
            pltpu.SemaphoreType.DMA,
        ],
    )
    def _sc_degree(src_hbm, out_hbm, src_v, ones_v, stripe_v, acc_sh, ssem):
        cid = lax.axis_index("c")
        sid = lax.axis_index("s")
        off = _stripe_off(sid)
        wid = cid * _NS + sid
        pltpu.sync_copy(src_hbm.at[wid], src_v)
        _zero_fill(stripe_v, _NPT, 16)
        pltpu.sync_copy(stripe_v, acc_sh.at[pl.ds(off, _NPT)])
        one = jnp.ones((16,), jnp.float32)

        def orow(i, _):
            ones_v[i, :] = one
            return 0

        lax.fori_loop(0, _CHUNK, orow, 0)
        plsc.subcore_barrier()

        def group(g, _):
            sd = [pltpu.async_copy(ones_v, acc_sh.at[src_v.at[g * _G + b]],
                                   ssem, add=True)
                  for b in range(_G)]
            for d in sd:
                d.wait()
            return 0

        lax.fori_loop(0, _NG, group, 0)
        plsc.subcore_barrier()
        pltpu.sync_copy(acc_sh.at[pl.ds(off, _NPT)],
                        out_hbm.at[cid, pl.ds(off, _NPT)])

    return _sc_degree


def _bn(z, eps=1e-5):
    m = jnp.mean(z, axis=0)
    v = jnp.mean((z - m) ** 2, axis=0)
    return (z - m) / jnp.sqrt(v + eps)


def _tc(body, *args, out_shape):
    return pl.pallas_call(body, out_shape=out_shape)(*args)


_MB = 1000


def _tc_dinv_body(degp_ref, dinv_ref):
    degp = degp_ref[...]
    deg = degp[0, :, 0] + degp[1, :, 0]
    dinv_ref[...] = jnp.where(
        deg > 0, lax.rsqrt(jnp.maximum(deg, 1.0)), 0.0)[:, None]


def _tc_proj1_body(x_ref, w1_ref, g1w_ref, dinv_ref, v_ref, xg_ref, c7_ref):
    x = x_ref[...]
    v1 = jnp.dot(x, w1_ref[...], preferred_element_type=jnp.float32)
    xg = jnp.dot(x, g1w_ref[...], preferred_element_type=jnp.float32)
    v_ref[...] = v1
    xg_ref[...] = xg
    c7_ref[...] = dinv_ref[...] * v1[:, 7 * _PAD:]


def _tc_proj2_body(x11_ref, w2_ref, dinv_ref, v_ref, c27_ref):
    v2 = jnp.dot(x11_ref[...], w2_ref[...], preferred_element_type=jnp.float32)
    v_ref[...] = v2
    c27_ref[...] = dinv_ref[...] * v2[:, 7 * _PAD:]


def _tc_comb1_body(parts_ref, partsg_ref, dinv_ref, v6_ref, xg_ref, g1b_ref,
                   g2w_ref, g2b_ref, b6_ref, c6_ref, x2_ref):
    p = parts_ref[...]
    pg = partsg_ref[...]
    dinv = dinv_ref[...]
    s = p[0] + p[1]
    agg = pg[0] + pg[1]
    b6 = v6_ref[...] - 2.0 * dinv * s
    b6_ref[...] = b6
    c6_ref[...] = dinv * b6
    h = jax.nn.relu(xg_ref[...] + agg + g1b_ref[...])
    g = jnp.dot(h, g2w_ref[...], preferred_element_type=jnp.float32) + g2b_ref[...]
    x2_ref[...] = _bn(jax.nn.relu(g))


def _tc_comb_body(parts_ref, dinv_ref, vprev_ref, bnext_ref, bout_ref, cout_ref):
    p = parts_ref[...]
    dinv = dinv_ref[...]
    b = vprev_ref[...] - 2.0 * dinv * (p[0] + p[1]) - bnext_ref[...]
    bout_ref[...] = b
    cout_ref[...] = dinv * b


def _tc_l1end_body(parts_ref, dinv_ref, v0_ref, b2_ref, bias_ref, x11_ref):
    p = parts_ref[...]
    dinv = dinv_ref[...]
    out1 = v0_ref[...] - dinv * (p[0] + p[1]) - b2_ref[...] + bias_ref[...]
    x11_ref[...] = jax.nn.relu(_bn(out1))


def _tc_l2end_body(parts_ref, dinv_ref, v0_ref, b2_ref, bias_ref, x11_ref,
                   x2_ref, w4_ref, b4_ref, y_ref):
    p = parts_ref[...]
    dinv = dinv_ref[...]
    out2 = v0_ref[...] - dinv * (p[0] + p[1]) - b2_ref[...] + bias_ref[...]
    x12 = jax.nn.relu(_bn(out2))
    x4 = jnp.concatenate(
        [x11_ref[...][:, :_NH], x12[:, :_NH], x2_ref[...][:, :_NH]], axis=1)
    y_ref[...] = jnp.dot(x4, w4_ref[...],
                         preferred_element_type=jnp.float32) + b4_ref[...]


def kernel(x, edge_index, W1, b1, W2, b2, G1w, G1b, G2w, G2b, W4, b4):
    f32 = jnp.float32
    src = edge_index[0]
    dst = edge_index[1]
    pw = _PAD - _NH

    W1c = jnp.pad(W1, ((0, 0), (0, 0), (0, pw))).transpose(1, 0, 2).reshape(_NF, _K * _PAD)
    W2c = jnp.pad(W2, ((0, 0), (0, pw), (0, pw))).transpose(1, 0, 2).reshape(_PAD, _K * _PAD)
    G1wp = jnp.pad(G1w, ((0, 0), (0, pw)))
    G2wp = jnp.pad(G2w, ((0, pw), (0, pw)))
    G1bp = jnp.pad(G1b, (0, pw))[None, :]
    G2bp = jnp.pad(G2b, (0, pw))[None, :]
    b1p = jnp.pad(b1, (0, pw))[None, :]
    b2p = jnp.pad(b2, (0, pw))[None, :]

    sds = jax.ShapeDtypeStruct
    nh48 = sds((_N, _PAD), f32)

    _prop48 = _make_prop(_PAD)
    src3 = src.reshape(_NW, _NCHUNK, _CHUNK)
    dst3 = dst.reshape(_NW, _NCHUNK, _CHUNK)
    degp = _make_degree()(src3)
    dinv = _tc(_tc_dinv_body, degp, out_shape=sds((_N, 1), f32))

    ng = _N // _MB
    rb = lambda w: pl.BlockSpec((_MB, w), lambda i: (i, 0))
    full = lambda a: pl.BlockSpec(a.shape, lambda i: (0, 0))
    V1, xg, c = pl.pallas_call(
        _tc_proj1_body, grid=(ng,),
        in_specs=[rb(_NF), full(W1c), full(G1wp), rb(1)],
        out_specs=(rb(_K * _PAD), rb(_PAD), rb(_PAD)),
        out_shape=(sds((_N, _K * _PAD), f32), nh48, nh48),
    )(x, W1c, G1wp, dinv)

    v = lambda V, k: lax.slice_in_dim(V, k * _PAD, (k + 1) * _PAD, axis=1)

    parts = _prop48(c, src3, dst3)
    partsg = _prop48(xg, src3, dst3)
    b6, c, x2 = _tc(_tc_comb1_body, parts, partsg, dinv, v(V1, 6), xg,
                    G1bp, G2wp, G2bp, out_shape=(nh48, nh48, nh48))
    bs = {7: v(V1, 7), 6: b6}
    for j in range(6, 1, -1):
        parts = _prop48(c, src3, dst3)
        bnew, c = _tc(_tc_comb_body, parts, dinv, v(V1, j - 1), bs[j + 1],
                      out_shape=(nh48, nh48))
        bs[j - 1] = bnew
    parts = _prop48(c, src3, dst3)
    x11 = _tc(_tc_l1end_body, parts, dinv, v(V1, 0), bs[2], b1p,
              out_shape=nh48)
    V2, c = pl.pallas_call(
        _tc_proj2_body, grid=(ng,),
        in_specs=[rb(_PAD), full(W2c), rb(1)],
        out_specs=(rb(_K * _PAD), rb(_PAD)),
        out_shape=(sds((_N, _K * _PAD), f32), nh48),
    )(x11, W2c, dinv)

    parts = _prop48(c, src3, dst3)
    b6, c = _tc(_tc_comb_body, parts, dinv, v(V2, 6), jnp.zeros((_N, _PAD), f32),
                out_shape=(nh48, nh48))
    bs2 = {7: v(V2, 7), 6: b6}
    for j in range(6, 1, -1):
        parts = _prop48(c, src3, dst3)
        bnew, c = _tc(_tc_comb_body, parts, dinv, v(V2, j - 1), bs2[j + 1],
                      out_shape=(nh48, nh48))
        bs2[j - 1] = bnew
    parts = _prop48(c, src3, dst3)
    y = _tc(_tc_l2end_body, parts, dinv, v(V2, 0), bs2[2], b2p, x11, x2,
            W4[0], b4[None, :], out_shape=sds((_N, _NCLS), f32))
    return y
